# Initial kernel scaffold; baseline (speedup 1.0000x reference)
import jax
import jax.numpy as jnp
from jax.experimental import pallas as pl


def kernel(x, edge_index, rel_type, W0, W1, W2, W3, Rw0, Rw1, Rw2, Rw3, b0, b1, b2, b3, lin1_W, lin1_b, lin2_W, lin2_b):
    raise NotImplementedError("write your pallas kernel here")


# speedup vs baseline: 32.6548x; 32.6548x over previous
import functools

import jax
import jax.numpy as jnp
from jax import lax
from jax.experimental import pallas as pl
from jax.experimental.pallas import tpu as pltpu
from jax.experimental.pallas import tpu_sc as plsc

N = 10000
E = 320000
IN = 128
LAT = 32
R = 5
B = 512

NC = 2
NS = 16
NW = NC * NS
CH = 80
NCH = 128
EC = CH * NCH
EPAD = NW * EC
NPAD = 10240
ROWS_PER_TILE = NPAD // NS

_f32 = jnp.float32


_sc_mesh = plsc.VectorSubcoreMesh(core_axis_name="c", subcore_axis_name="s")


@functools.partial(
    pl.kernel,
    mesh=_sc_mesh,
    compiler_params=pltpu.CompilerParams(use_tc_tiling_on_sc=False),
    out_type=jax.ShapeDtypeStruct((NC, NPAD, LAT), _f32),
    scratch_types=[
        pltpu.VMEM((NCH, CH), jnp.int32),
        pltpu.VMEM((NCH, CH), jnp.int32),
        pltpu.VMEM((CH, LAT), _f32),
        pltpu.VMEM((CH, LAT), _f32),
        pltpu.VMEM((ROWS_PER_TILE, LAT), _f32),
        pltpu.VMEM_SHARED((NPAD, LAT), _f32),
        pltpu.SemaphoreType.DMA,
        pltpu.SemaphoreType.DMA,
    ],
)
def _sc_edge_aggregate(t_hbm, gidx_hbm, dst_hbm, out_hbm,
                       gidx_v, dst_v, rows0, rows1, zbuf, o_sh,
                       sem0, sem1):
    c = lax.axis_index("c")
    s = lax.axis_index("s")
    wid = s * NC + c

    def _zrow(i, carry):
        zbuf[i, pl.ds(0, 16)] = jnp.zeros((16,), _f32)
        zbuf[i, pl.ds(16, 16)] = jnp.zeros((16,), _f32)
        return carry

    lax.fori_loop(0, ROWS_PER_TILE, _zrow, 0)
    pltpu.sync_copy(zbuf, o_sh.at[pl.ds(s * ROWS_PER_TILE, ROWS_PER_TILE)])
    plsc.subcore_barrier()

    pltpu.sync_copy(gidx_hbm.at[wid], gidx_v)
    pltpu.sync_copy(dst_hbm.at[wid], dst_v)

    pltpu.async_copy(t_hbm.at[gidx_v.at[0]], rows0, sem0)

    def _pair(jj, carry):
        j0 = jj * 2
        pltpu.async_copy(t_hbm.at[gidx_v.at[j0 + 1]], rows1, sem1)
        pltpu.make_async_copy(t_hbm.at[gidx_v.at[j0]], rows0, sem0).wait()
        pltpu.sync_copy(rows0, o_sh.at[dst_v.at[j0]], add=True)

        @pl.when(j0 + 2 < NCH)
        def _():
            pltpu.async_copy(t_hbm.at[gidx_v.at[j0 + 2]], rows0, sem0)

        pltpu.make_async_copy(t_hbm.at[gidx_v.at[j0 + 1]], rows1, sem1).wait()
        pltpu.sync_copy(rows1, o_sh.at[dst_v.at[j0 + 1]], add=True)
        return carry

    lax.fori_loop(0, NCH // 2, _pair, 0)

    plsc.subcore_barrier()
    pltpu.sync_copy(
        o_sh.at[pl.ds(s * ROWS_PER_TILE, ROWS_PER_TILE)],
        out_hbm.at[c, pl.ds(s * ROWS_PER_TILE, ROWS_PER_TILE)],
    )


def _tc_trans_body(h_ref, wf_ref, rw_ref, b_ref, t_ref, z_ref):
    h = h_ref[...]
    t_ref[...] = jnp.dot(h, wf_ref[...], preferred_element_type=_f32)
    z_ref[...] = jnp.dot(h, rw_ref[...], preferred_element_type=_f32) + b_ref[...]


def _tc_trans(h, wf, rw, b):
    n, _ = h.shape
    return pl.pallas_call(
        _tc_trans_body,
        out_shape=[
            jax.ShapeDtypeStruct((n, R * LAT), _f32),
            jax.ShapeDtypeStruct((n, LAT), _f32),
        ],
    )(h, wf, rw, b)


def _tc_comb_trans_body(z_ref, o_ref, wf_ref, rw_ref, b_ref,
                        h_ref, t_ref, z2_ref):
    o = o_ref[0, :N, :] + o_ref[1, :N, :]
    h = jnp.tanh(z_ref[...] + o)
    h_ref[...] = h
    t_ref[...] = jnp.dot(h, wf_ref[...], preferred_element_type=_f32)
    z2_ref[...] = jnp.dot(h, rw_ref[...], preferred_element_type=_f32) + b_ref[...]


def _tc_comb_trans(z, o, wf, rw, b):
    return pl.pallas_call(
        _tc_comb_trans_body,
        out_shape=[
            jax.ShapeDtypeStruct((N, LAT), _f32),
            jax.ShapeDtypeStruct((N, R * LAT), _f32),
            jax.ShapeDtypeStruct((N, LAT), _f32),
        ],
    )(z, o, wf, rw, b)


def _tc_final_body(z_ref, o_ref, h1_ref, h2_ref, h3_ref,
                   l1w_ref, l1b_ref, l2w_ref, l2b_ref, out_ref):
    h4 = jnp.tanh(z_ref[...] + o_ref[0, :2 * B, :] + o_ref[1, :2 * B, :])
    hcat = jnp.concatenate([h1_ref[...], h2_ref[...], h3_ref[...], h4], axis=1)
    u = jnp.concatenate([hcat[:B], hcat[B:]], axis=1)
    a = jnp.dot(u, l1w_ref[...], preferred_element_type=_f32) + l1b_ref[...]
    a = jnp.maximum(a, 0.0)
    out_ref[...] = jnp.dot(a, l2w_ref[...], preferred_element_type=_f32) + l2b_ref[...]


def _tc_final(z, o, h1, h2, h3, l1w, l1b, l2w, l2b):
    return pl.pallas_call(
        _tc_final_body,
        out_shape=jax.ShapeDtypeStruct((B, 1), _f32),
    )(z, o, h1, h2, h3, l1w, l1b, l2w, l2b)


def kernel(x, edge_index, rel_type,
           W0, W1, W2, W3,
           Rw0, Rw1, Rw2, Rw3,
           b0, b1, b2, b3,
           lin1_W, lin1_b, lin2_W, lin2_b):
    src = edge_index[0].astype(jnp.int32)
    dst = edge_index[1].astype(jnp.int32)
    gidx = src * R + rel_type.astype(jnp.int32)
    pad = EPAD - E
    gidx_p = jnp.concatenate([gidx, jnp.zeros((pad,), jnp.int32)])
    dst_p = jnp.concatenate([dst, jnp.full((pad,), NPAD - 1, jnp.int32)])
    gidx3 = gidx_p.reshape(NW, NCH, CH)
    dst3 = dst_p.reshape(NW, NCH, CH)

    wfs = [jnp.transpose(W, (1, 0, 2)).reshape(W.shape[1], R * LAT)
           for W in (W0, W1, W2, W3)]
    rws = (Rw0, Rw1, Rw2, Rw3)
    bs = [b.reshape(1, LAT) for b in (b0, b1, b2, b3)]

    t, z = _tc_trans(x, wfs[0], rws[0], bs[0])
    hs = []
    for l in range(1, 4):
        o = _sc_edge_aggregate(t.reshape(N * R, LAT), gidx3, dst3)
        h, t, z = _tc_comb_trans(z, o, wfs[l], rws[l], bs[l])
        hs.append(h)
    o = _sc_edge_aggregate(t.reshape(N * R, LAT), gidx3, dst3)
    out = _tc_final(z[:2 * B], o,
                    hs[0][:2 * B], hs[1][:2 * B], hs[2][:2 * B],
                    lin1_W, lin1_b.reshape(1, 128),
                    lin2_W, lin2_b.reshape(1, 1))
    return out[:, 0]
